# baseline (speedup 1.0000x reference)
import jax
import jax.numpy as jnp
from jax.experimental import pallas as pl
from jax.experimental.pallas import tpu as pltpu

_BR = 4000


def _copy_body(src_ref, dst_ref):
    dst_ref[...] = src_ref[...]


def kernel(embeddings):
    rows, dim = embeddings.shape
    v = embeddings.reshape(rows // 2, dim * 2)
    grid = (rows // 2) // _BR
    out = pl.pallas_call(
        _copy_body,
        out_shape=jax.ShapeDtypeStruct(v.shape, v.dtype),
        grid=(grid,),
        in_specs=[pl.BlockSpec((_BR, dim * 2), lambda i: (i, 0))],
        out_specs=pl.BlockSpec((_BR, dim * 2), lambda i: (i, 0)),
    )(v)
    return out.reshape(rows, dim)


# speedup vs baseline: 1.8110x; 1.8110x over previous
import jax
import jax.numpy as jnp
from jax.experimental import pallas as pl
from jax.experimental.pallas import tpu as pltpu

_BR = 1000
_N = 125
_NBUF = 12
_LOOKAHEAD = 6


def _copy_body(src, dst, buf, in_sems, out_sems):
    def in_cp(i):
        return pltpu.make_async_copy(
            src.at[:, pl.ds(i * _BR, _BR), :], buf.at[i % _NBUF],
            in_sems.at[i % _NBUF])

    def out_cp(i):
        return pltpu.make_async_copy(
            buf.at[i % _NBUF], dst.at[:, pl.ds(i * _BR, _BR), :],
            out_sems.at[i % _NBUF])

    for i in range(_LOOKAHEAD):
        in_cp(i).start()
    for i in range(_N):
        in_cp(i).wait()
        out_cp(i).start()
        nxt = i + _LOOKAHEAD
        if nxt < _N:
            if nxt >= _NBUF:
                out_cp(nxt - _NBUF).wait()
            in_cp(nxt).start()
    for i in range(max(0, _N - _NBUF), _N):
        out_cp(i).wait()


def kernel(embeddings):
    rows, dim = embeddings.shape
    v = embeddings.reshape(8, rows // 8, dim)
    out = pl.pallas_call(
        _copy_body,
        out_shape=jax.ShapeDtypeStruct(v.shape, v.dtype),
        in_specs=[pl.BlockSpec(memory_space=pl.ANY)],
        out_specs=pl.BlockSpec(memory_space=pl.ANY),
        scratch_shapes=[
            pltpu.VMEM((_NBUF, 8, _BR, dim), v.dtype),
            pltpu.SemaphoreType.DMA((_NBUF,)),
            pltpu.SemaphoreType.DMA((_NBUF,)),
        ],
    )(v)
    return out.reshape(rows, dim)
